# baseline (speedup 1.0000x reference)
import functools

import jax
import jax.numpy as jnp
from jax import lax
from jax.experimental import pallas as pl
from jax.experimental.pallas import tpu as pltpu
from jax.experimental.pallas import tpu_sc as plsc

N = 10000
E = 320000
D = 128
DE = 16

NC = 2
NS = 16
NW = NC * NS
EROWS = E // 128
ERPAD = 2560
WROWS = 80
NROWS = 79
NPAD = NROWS * 128
EBLK = 2048
ERB = 128
NSTEPS = 20


def _node_body(h_ref, wn_ref, wan_ref, nf_ref, ns_ref):
    nf = lax.dot_general(h_ref[...], wn_ref[...], (((1,), (1,)), ((), ())),
                         preferred_element_type=jnp.float32)
    nf_ref[...] = nf
    ns_ref[...] = jnp.dot(nf, wan_ref[...], preferred_element_type=jnp.float32)


def _edge_body(e_ref, wbig_ref, wat_ref, src1_ref, ns_ref,
               ew_ref, es_ref, srcr_ref, nsr_ref):
    ew = jnp.dot(e_ref[...], wbig_ref[...], preferred_element_type=jnp.float32)
    ew_ref[...] = ew
    row = lax.broadcasted_iota(jnp.int32, (128, 8), 0)
    col = lax.broadcasted_iota(jnp.int32, (128, 8), 1)
    g = (row // DE == col).astype(jnp.float32)
    es8 = jnp.dot(ew * wat_ref[...], g, preferred_element_type=jnp.float32)
    es3 = es8.reshape(ERB, 16, 8)
    es128 = jnp.concatenate([es3[:, a, :] for a in range(16)], axis=1)
    gid = pl.program_id(0)
    rows = lax.broadcasted_iota(jnp.int32, (ERB, 128), 0) + gid * ERB
    valid = rows < EROWS
    es_ref[...] = jnp.where(valid, es128, 0.0)
    srcr_ref[...] = jnp.where(valid, src1_ref[...], 0)
    nsp = jnp.concatenate([ns_ref[...][:, 0],
                           jnp.zeros((NPAD - N,), jnp.float32)])
    nsr_ref[...] = nsp.reshape(NROWS, 128)


def _scale_body(ew_ref, g_ref, out_ref):
    row = lax.broadcasted_iota(jnp.int32, (8, 128), 0)
    col = lax.broadcasted_iota(jnp.int32, (8, 128), 1)
    gt = (col // DE == row).astype(jnp.float32)
    g128 = g_ref[...]
    g8 = jnp.stack([g128[:, 8 * a:8 * a + 8] for a in range(16)],
                   axis=1).reshape(EBLK, 8)
    gexp = jnp.dot(g8, gt, preferred_element_type=jnp.float32)
    out_ref[...] = ew_ref[...] * gexp


def _worker_rows(wid):
    raw = (wid * EROWS) // NW
    base = ((raw + 7) // 8) * 8
    raw_n = ((wid + 1) * EROWS) // NW
    nxt = jnp.where(wid == NW - 1, EROWS, ((raw_n + 7) // 8) * 8)
    return base, nxt - base


def _sc1_body(src_hbm, es_hbm, ns_hbm, ex_hbm, spart_hbm,
              src_v, es_v, ns_v, ex_v, zb_v, s_sh, dsem, xsem):
    c = lax.axis_index("c")
    s = lax.axis_index("s")
    wid = s * NC + c
    base, count = _worker_rows(wid)
    base = pl.multiple_of(base, 8)

    pltpu.sync_copy(src_hbm.at[pl.ds(base, WROWS)], src_v)
    pltpu.sync_copy(es_hbm.at[pl.ds(base, WROWS)], es_v)
    pltpu.sync_copy(ns_hbm, ns_v)

    @pl.when(s == 0)
    def _():
        zeros16 = jnp.zeros((16,), jnp.float32)

        @plsc.parallel_loop(0, NPAD // 16, unroll=4)
        def _(i):
            zb_v[pl.ds(pl.multiple_of(i * 16, 16), 16)] = zeros16
        pltpu.sync_copy(zb_v, s_sh)

    @plsc.parallel_loop(0, WROWS)
    def _(r):
        for k in range(8):
            sv = src_v[r, pl.ds(k * 16, 16)]
            nsv = plsc.load_gather(
                ns_v, [lax.shift_right_logical(sv, 7), lax.bitwise_and(sv, 127)])
            a = es_v[r, pl.ds(k * 16, 16)] + nsv
            eact = jnp.where(a >= 0.0, a, a * 0.01)
            ex_v[r, pl.ds(k * 16, 16)] = jnp.exp(eact)

    exd = pltpu.async_copy(ex_v, ex_hbm.at[pl.ds(base, WROWS)], xsem)

    plsc.subcore_barrier()

    def sgroup(gidx, _):
        for t in range(5):
            j = gidx * 5 + t

            @pl.when(j < count)
            def _():
                pltpu.async_copy(ex_v.at[j], s_sh.at[src_v.at[j]], dsem,
                                 add=True)
        for t in range(5):
            j = gidx * 5 + t

            @pl.when(j < count)
            def _():
                pltpu.make_async_copy(ex_v.at[j], s_sh.at[src_v.at[j]],
                                      dsem).wait()
        return ()
    lax.fori_loop(0, WROWS // 5, sgroup, ())

    plsc.subcore_barrier()
    exd.wait()

    @pl.when(s == 0)
    def _():
        pltpu.sync_copy(s_sh, spart_hbm.at[pl.ds(c * NPAD, NPAD)])


def _sc2_body(src_hbm, ex_hbm, spart_hbm, gamma_hbm,
              src_v, ex_v, s0_v, s1_v, g_v):
    c = lax.axis_index("c")
    s = lax.axis_index("s")
    wid = s * NC + c
    base, _count = _worker_rows(wid)
    base = pl.multiple_of(base, 8)

    pltpu.sync_copy(src_hbm.at[pl.ds(base, WROWS)], src_v)
    pltpu.sync_copy(ex_hbm.at[pl.ds(base, WROWS)], ex_v)
    pltpu.sync_copy(spart_hbm.at[pl.ds(0, NPAD)], s0_v)
    pltpu.sync_copy(spart_hbm.at[pl.ds(NPAD, NPAD)], s1_v)

    @plsc.parallel_loop(0, NPAD // 16, unroll=4)
    def _(i):
        off = pl.ds(pl.multiple_of(i * 16, 16), 16)
        s0_v[off] = s0_v[off] + s1_v[off]

    @plsc.parallel_loop(0, WROWS)
    def _(r):
        for k in range(8):
            sv = src_v[r, pl.ds(k * 16, 16)]
            st = plsc.load_gather(s0_v, [sv])
            g_v[r, pl.ds(k * 16, 16)] = ex_v[r, pl.ds(k * 16, 16)] / st

    pltpu.sync_copy(g_v, gamma_hbm.at[pl.ds(base, WROWS)])


@jax.jit
def kernel(h, e, edge_index, Wn, We, Wa):
    f32 = jnp.float32
    wa_n = Wa[0, DE:].reshape(D, 1)
    w_big = jnp.kron(jnp.eye(8, dtype=f32), We.T)
    wa_t = jnp.tile(Wa[0, :DE], 8).reshape(1, 128)

    nblk = 1000
    node_feat, n_score = pl.pallas_call(
        _node_body,
        grid=(N // nblk,),
        in_specs=[
            pl.BlockSpec((nblk, D), lambda i: (i, 0)),
            pl.BlockSpec((D, D), lambda i: (0, 0)),
            pl.BlockSpec((D, 1), lambda i: (0, 0)),
        ],
        out_specs=[
            pl.BlockSpec((nblk, D), lambda i: (i, 0)),
            pl.BlockSpec((nblk, 1), lambda i: (i, 0)),
        ],
        out_shape=[
            jax.ShapeDtypeStruct((N, D), f32),
            jax.ShapeDtypeStruct((N, 1), f32),
        ],
    )(h, Wn, wa_n)

    e128 = e.reshape(E * DE // 128, 128)
    src1 = edge_index[0].reshape(EROWS, 128)
    ew128, es128, src128, ns128 = pl.pallas_call(
        _edge_body,
        grid=(NSTEPS,),
        in_specs=[
            pl.BlockSpec((EBLK, 128), lambda i: (i, 0)),
            pl.BlockSpec((128, 128), lambda i: (0, 0)),
            pl.BlockSpec((1, 128), lambda i: (0, 0)),
            pl.BlockSpec((ERB, 128), lambda i: (i, 0)),
            pl.BlockSpec((N, 1), lambda i: (0, 0)),
        ],
        out_specs=[
            pl.BlockSpec((EBLK, 128), lambda i: (i, 0)),
            pl.BlockSpec((ERB, 128), lambda i: (i, 0)),
            pl.BlockSpec((ERB, 128), lambda i: (i, 0)),
            pl.BlockSpec((NROWS, 128), lambda i: (0, 0)),
        ],
        out_shape=[
            jax.ShapeDtypeStruct((e128.shape[0], 128), f32),
            jax.ShapeDtypeStruct((ERPAD, 128), f32),
            jax.ShapeDtypeStruct((ERPAD, 128), jnp.int32),
            jax.ShapeDtypeStruct((NROWS, 128), f32),
        ],
    )(e128, w_big, wa_t, src1, n_score)

    mesh = plsc.VectorSubcoreMesh(core_axis_name="c", subcore_axis_name="s")
    sc_params = pltpu.CompilerParams(needs_layout_passes=False)

    sc1 = pl.kernel(
        _sc1_body,
        compiler_params=sc_params,
        out_type=(
            jax.ShapeDtypeStruct((ERPAD, 128), f32),
            jax.ShapeDtypeStruct((NC * NPAD,), f32),
        ),
        mesh=mesh,
        scratch_types=[
            pltpu.VMEM((WROWS, 128), jnp.int32),
            pltpu.VMEM((WROWS, 128), f32),
            pltpu.VMEM((NROWS, 128), f32),
            pltpu.VMEM((WROWS, 128), f32),
            pltpu.VMEM((NPAD,), f32),
            pltpu.VMEM_SHARED((NPAD,), f32),
            pltpu.SemaphoreType.DMA,
            pltpu.SemaphoreType.DMA,
        ],
    )
    ex128, s_part = sc1(src128, es128, ns128)

    sc2 = pl.kernel(
        _sc2_body,
        compiler_params=sc_params,
        out_type=jax.ShapeDtypeStruct((ERPAD, 128), f32),
        mesh=mesh,
        scratch_types=[
            pltpu.VMEM((WROWS, 128), jnp.int32),
            pltpu.VMEM((WROWS, 128), f32),
            pltpu.VMEM((NPAD,), f32),
            pltpu.VMEM((NPAD,), f32),
            pltpu.VMEM((WROWS, 128), f32),
        ],
    )
    gamma128 = sc2(src128, ex128, s_part)

    ewt128 = pl.pallas_call(
        _scale_body,
        grid=(NSTEPS,),
        in_specs=[
            pl.BlockSpec((EBLK, 128), lambda i: (i, 0)),
            pl.BlockSpec((ERB, 128), lambda i: (i, 0)),
        ],
        out_specs=pl.BlockSpec((EBLK, 128), lambda i: (i, 0)),
        out_shape=jax.ShapeDtypeStruct((e128.shape[0], 128), f32),
    )(ew128, gamma128)

    e_weighted = ewt128.reshape(E, DE)
    return (node_feat, e_weighted)


# speedup vs baseline: 1.1276x; 1.1276x over previous
import functools

import jax
import jax.numpy as jnp
from jax import lax
from jax.experimental import pallas as pl
from jax.experimental.pallas import tpu as pltpu
from jax.experimental.pallas import tpu_sc as plsc

N = 10000
E = 320000
D = 128
DE = 16

NC = 2
NS = 16
NW = NC * NS
EROWS = E // 128
ERPAD = 2560
WROWS = 80
NROWS = 79
NPAD = NROWS * 128
EBLK = 2048
ERB = 128
NSTEPS = 20


def _node_body(h_ref, wn_ref, wan_ref, nf_ref, ns_ref):
    nf = lax.dot_general(h_ref[...], wn_ref[...], (((1,), (1,)), ((), ())),
                         preferred_element_type=jnp.float32)
    nf_ref[...] = nf
    ns_ref[...] = jnp.dot(nf, wan_ref[...], preferred_element_type=jnp.float32)


def _edge_body(e_ref, wbig_ref, wat_ref, g_ref, src1_ref, ns_ref,
               ew_ref, es_ref, srcr_ref, nsr_ref):
    ew = jnp.dot(e_ref[...], wbig_ref[...], preferred_element_type=jnp.float32)
    ew_ref[...] = ew
    es8 = jnp.dot(ew * wat_ref[...], g_ref[...],
                  preferred_element_type=jnp.float32)
    gid = pl.program_id(0)

    @pl.when(gid < NSTEPS - 1)
    def _():
        es_ref[...] = es8
        srcr_ref[...] = src1_ref[...]

    @pl.when(gid == NSTEPS - 1)
    def _():
        erow = lax.broadcasted_iota(jnp.int32, (EBLK, 8), 0) + gid * EBLK
        es_ref[...] = jnp.where(erow < E // 8, es8, 0.0)
        rows = lax.broadcasted_iota(jnp.int32, (ERB, 128), 0) + gid * ERB
        srcr_ref[...] = jnp.where(rows < EROWS, src1_ref[...], 0)

    @pl.when(gid == 0)
    def _():
        nsp = jnp.concatenate([ns_ref[...][:, 0],
                               jnp.zeros((NPAD - N,), jnp.float32)])
        nsr_ref[...] = nsp.reshape(NROWS, 128)


def _scale_body(ew_ref, g8_ref, gt_ref, out_ref):
    gexp = jnp.dot(g8_ref[...], gt_ref[...], preferred_element_type=jnp.float32)
    out_ref[...] = ew_ref[...] * gexp


def _worker_rows(wid):
    raw = (wid * EROWS) // NW
    base = ((raw + 7) // 8) * 8
    raw_n = ((wid + 1) * EROWS) // NW
    nxt = jnp.where(wid == NW - 1, EROWS, ((raw_n + 7) // 8) * 8)
    return base, nxt - base


def _sc1_body(src_hbm, es_hbm, ns_hbm, ex_hbm, spart_hbm,
              src_v, es_v, ns_v, ex_v, zb_v, s_sh, dsem, xsem):
    c = lax.axis_index("c")
    s = lax.axis_index("s")
    wid = s * NC + c
    base, count = _worker_rows(wid)
    base = pl.multiple_of(base, 8)

    pltpu.sync_copy(src_hbm.at[pl.ds(base, WROWS)], src_v)
    pltpu.sync_copy(es_hbm.at[pl.ds(base, WROWS)], es_v)
    pltpu.sync_copy(ns_hbm, ns_v)

    @pl.when(s == 0)
    def _():
        zeros16 = jnp.zeros((16,), jnp.float32)

        @plsc.parallel_loop(0, NPAD // 16, unroll=4)
        def _(i):
            zb_v[pl.ds(pl.multiple_of(i * 16, 16), 16)] = zeros16
        pltpu.sync_copy(zb_v, s_sh)

    @plsc.parallel_loop(0, WROWS)
    def _(r):
        for k in range(8):
            sv = src_v[r, pl.ds(k * 16, 16)]
            nsv = plsc.load_gather(
                ns_v, [lax.shift_right_logical(sv, 7), lax.bitwise_and(sv, 127)])
            a = es_v[r, pl.ds(k * 16, 16)] + nsv
            eact = jnp.where(a >= 0.0, a, a * 0.01)
            ex_v[r, pl.ds(k * 16, 16)] = jnp.exp(eact)

    exd = pltpu.async_copy(ex_v, ex_hbm.at[pl.ds(base, WROWS)], xsem)

    plsc.subcore_barrier()

    def sgroup(gidx, _):
        for t in range(5):
            j = gidx * 5 + t

            @pl.when(j < count)
            def _():
                pltpu.async_copy(ex_v.at[j], s_sh.at[src_v.at[j]], dsem,
                                 add=True)
        for t in range(5):
            j = gidx * 5 + t

            @pl.when(j < count)
            def _():
                pltpu.make_async_copy(ex_v.at[j], s_sh.at[src_v.at[j]],
                                      dsem).wait()
        return ()
    lax.fori_loop(0, WROWS // 5, sgroup, ())

    plsc.subcore_barrier()
    exd.wait()

    @pl.when(s == 0)
    def _():
        pltpu.sync_copy(s_sh, spart_hbm.at[pl.ds(c * NPAD, NPAD)])


def _sc2_body(src_hbm, ex_hbm, spart_hbm, gamma_hbm,
              src_v, ex_v, s0_v, s1_v, g_v):
    c = lax.axis_index("c")
    s = lax.axis_index("s")
    wid = s * NC + c
    base, _count = _worker_rows(wid)
    base = pl.multiple_of(base, 8)

    pltpu.sync_copy(src_hbm.at[pl.ds(base, WROWS)], src_v)
    pltpu.sync_copy(ex_hbm.at[pl.ds(base, WROWS)], ex_v)
    pltpu.sync_copy(spart_hbm.at[pl.ds(0, NPAD)], s0_v)
    pltpu.sync_copy(spart_hbm.at[pl.ds(NPAD, NPAD)], s1_v)

    @plsc.parallel_loop(0, NPAD // 16, unroll=4)
    def _(i):
        off = pl.ds(pl.multiple_of(i * 16, 16), 16)
        s0_v[off] = s0_v[off] + s1_v[off]

    @plsc.parallel_loop(0, WROWS)
    def _(r):
        for k in range(8):
            sv = src_v[r, pl.ds(k * 16, 16)]
            st = plsc.load_gather(s0_v, [sv])
            g_v[r, pl.ds(k * 16, 16)] = ex_v[r, pl.ds(k * 16, 16)] / st

    pltpu.sync_copy(g_v, gamma_hbm.at[pl.ds(base, WROWS)])


@jax.jit
def kernel(h, e, edge_index, Wn, We, Wa):
    f32 = jnp.float32
    wa_n = Wa[0, DE:].reshape(D, 1)
    w_big = jnp.kron(jnp.eye(8, dtype=f32), We.T)
    wa_t = jnp.tile(Wa[0, :DE], 8).reshape(1, 128)

    nblk = 1000
    node_feat, n_score = pl.pallas_call(
        _node_body,
        grid=(N // nblk,),
        in_specs=[
            pl.BlockSpec((nblk, D), lambda i: (i, 0)),
            pl.BlockSpec((D, D), lambda i: (0, 0)),
            pl.BlockSpec((D, 1), lambda i: (0, 0)),
        ],
        out_specs=[
            pl.BlockSpec((nblk, D), lambda i: (i, 0)),
            pl.BlockSpec((nblk, 1), lambda i: (i, 0)),
        ],
        out_shape=[
            jax.ShapeDtypeStruct((N, D), f32),
            jax.ShapeDtypeStruct((N, 1), f32),
        ],
    )(h, Wn, wa_n)

    e128 = e.reshape(E * DE // 128, 128)
    src1 = edge_index[0].reshape(EROWS, 128)
    gsel = jnp.repeat(jnp.eye(8, dtype=f32), DE, axis=0)
    ew128, es8a, src128, ns128 = pl.pallas_call(
        _edge_body,
        grid=(NSTEPS,),
        in_specs=[
            pl.BlockSpec((EBLK, 128), lambda i: (i, 0)),
            pl.BlockSpec((128, 128), lambda i: (0, 0)),
            pl.BlockSpec((1, 128), lambda i: (0, 0)),
            pl.BlockSpec((128, 8), lambda i: (0, 0)),
            pl.BlockSpec((ERB, 128), lambda i: (i, 0)),
            pl.BlockSpec((N, 1), lambda i: (0, 0)),
        ],
        out_specs=[
            pl.BlockSpec((EBLK, 128), lambda i: (i, 0)),
            pl.BlockSpec((EBLK, 8), lambda i: (i, 0)),
            pl.BlockSpec((ERB, 128), lambda i: (i, 0)),
            pl.BlockSpec((NROWS, 128), lambda i: (0, 0)),
        ],
        out_shape=[
            jax.ShapeDtypeStruct((e128.shape[0], 128), f32),
            jax.ShapeDtypeStruct((ERPAD * 16, 8), f32),
            jax.ShapeDtypeStruct((ERPAD, 128), jnp.int32),
            jax.ShapeDtypeStruct((NROWS, 128), f32),
        ],
    )(e128, w_big, wa_t, gsel, src1, n_score)
    es128 = es8a.reshape(ERPAD, 128)

    mesh = plsc.VectorSubcoreMesh(core_axis_name="c", subcore_axis_name="s")
    sc_params = pltpu.CompilerParams(needs_layout_passes=False)

    sc1 = pl.kernel(
        _sc1_body,
        compiler_params=sc_params,
        out_type=(
            jax.ShapeDtypeStruct((ERPAD, 128), f32),
            jax.ShapeDtypeStruct((NC * NPAD,), f32),
        ),
        mesh=mesh,
        scratch_types=[
            pltpu.VMEM((WROWS, 128), jnp.int32),
            pltpu.VMEM((WROWS, 128), f32),
            pltpu.VMEM((NROWS, 128), f32),
            pltpu.VMEM((WROWS, 128), f32),
            pltpu.VMEM((NPAD,), f32),
            pltpu.VMEM_SHARED((NPAD,), f32),
            pltpu.SemaphoreType.DMA,
            pltpu.SemaphoreType.DMA,
        ],
    )
    ex128, s_part = sc1(src128, es128, ns128)

    sc2 = pl.kernel(
        _sc2_body,
        compiler_params=sc_params,
        out_type=jax.ShapeDtypeStruct((ERPAD, 128), f32),
        mesh=mesh,
        scratch_types=[
            pltpu.VMEM((WROWS, 128), jnp.int32),
            pltpu.VMEM((WROWS, 128), f32),
            pltpu.VMEM((NPAD,), f32),
            pltpu.VMEM((NPAD,), f32),
            pltpu.VMEM((WROWS, 128), f32),
        ],
    )
    gamma128 = sc2(src128, ex128, s_part)

    gtile = jnp.repeat(jnp.eye(8, dtype=f32), DE, axis=1)
    ewt128 = pl.pallas_call(
        _scale_body,
        grid=(NSTEPS,),
        in_specs=[
            pl.BlockSpec((EBLK, 128), lambda i: (i, 0)),
            pl.BlockSpec((EBLK, 8), lambda i: (i, 0)),
            pl.BlockSpec((8, 128), lambda i: (0, 0)),
        ],
        out_specs=pl.BlockSpec((EBLK, 128), lambda i: (i, 0)),
        out_shape=jax.ShapeDtypeStruct((e128.shape[0], 128), f32),
    )(ew128, gamma128.reshape(ERPAD * 16, 8), gtile)

    e_weighted = ewt128.reshape(E, DE)
    return (node_feat, e_weighted)


# speedup vs baseline: 1.1476x; 1.0177x over previous
import functools

import jax
import jax.numpy as jnp
from jax import lax
from jax.experimental import pallas as pl
from jax.experimental.pallas import tpu as pltpu
from jax.experimental.pallas import tpu_sc as plsc

N = 10000
E = 320000
D = 128
DE = 16

NC = 2
NS = 16
NW = NC * NS
EROWS = E // 128
ERPAD = 2560
WROWS = 80
NROWS = 79
NPAD = NROWS * 128
EBLK = 2048
ERB = 128
NSTEPS = 20


def _edge_body(h_ref, wn_ref, wan_ref, e_ref, wbig_ref, wat_ref, g_ref,
               src1_ref, nf_ref, ns_ref, ew_ref, es_ref, srcr_ref):
    nf = lax.dot_general(h_ref[...], wn_ref[...], (((1,), (1,)), ((), ())),
                         preferred_element_type=jnp.float32)
    nf_ref[...] = nf
    ns_ref[...] = jnp.dot(nf, wan_ref[...], preferred_element_type=jnp.float32)

    ew = jnp.dot(e_ref[...], wbig_ref[...], preferred_element_type=jnp.float32)
    ew_ref[...] = ew
    es8 = jnp.dot(ew * wat_ref[...], g_ref[...],
                  preferred_element_type=jnp.float32)
    gid = pl.program_id(0)

    @pl.when(gid < NSTEPS - 1)
    def _():
        es_ref[...] = es8
        srcr_ref[...] = src1_ref[...]

    @pl.when(gid == NSTEPS - 1)
    def _():
        erow = lax.broadcasted_iota(jnp.int32, (EBLK, 8), 0) + gid * EBLK
        es_ref[...] = jnp.where(erow < E // 8, es8, 0.0)
        rows = lax.broadcasted_iota(jnp.int32, (ERB, 128), 0) + gid * ERB
        srcr_ref[...] = jnp.where(rows < EROWS, src1_ref[...], 0)


def _scale_body(ew_ref, g8_ref, gt_ref, out_ref):
    gexp = jnp.dot(g8_ref[...], gt_ref[...], preferred_element_type=jnp.float32)
    out_ref[...] = ew_ref[...] * gexp


def _worker_rows(wid):
    raw = (wid * EROWS) // NW
    base = ((raw + 7) // 8) * 8
    raw_n = ((wid + 1) * EROWS) // NW
    nxt = jnp.where(wid == NW - 1, EROWS, ((raw_n + 7) // 8) * 8)
    return base, nxt - base


def _sc1_body(src_hbm, es_hbm, ns_hbm, ex_hbm, spart_hbm,
              src_v, es_v, ns_v, ex_v, zb_v, s_sh, dsem, xsem):
    c = lax.axis_index("c")
    s = lax.axis_index("s")
    wid = s * NC + c
    base, count = _worker_rows(wid)
    base = pl.multiple_of(base, 8)

    pltpu.sync_copy(src_hbm.at[pl.ds(base, WROWS)], src_v)
    pltpu.sync_copy(es_hbm.at[pl.ds(base, WROWS)], es_v)
    pltpu.sync_copy(ns_hbm, ns_v)

    @pl.when(s == 0)
    def _():
        zeros16 = jnp.zeros((16,), jnp.float32)

        @plsc.parallel_loop(0, NPAD // 16, unroll=4)
        def _(i):
            zb_v[pl.ds(pl.multiple_of(i * 16, 16), 16)] = zeros16
        pltpu.sync_copy(zb_v, s_sh)

    @plsc.parallel_loop(0, WROWS)
    def _(r):
        for k in range(8):
            sv = src_v[r, pl.ds(k * 16, 16)]
            nsv = plsc.load_gather(ns_v, [sv])
            a = es_v[r, pl.ds(k * 16, 16)] + nsv
            eact = jnp.where(a >= 0.0, a, a * 0.01)
            ex_v[r, pl.ds(k * 16, 16)] = jnp.exp(eact)

    exd = pltpu.async_copy(ex_v, ex_hbm.at[pl.ds(base, WROWS)], xsem)

    plsc.subcore_barrier()

    def sgroup(gidx, _):
        for t in range(5):
            j = gidx * 5 + t

            @pl.when(j < count)
            def _():
                pltpu.async_copy(ex_v.at[j], s_sh.at[src_v.at[j]], dsem,
                                 add=True)
        for t in range(5):
            j = gidx * 5 + t

            @pl.when(j < count)
            def _():
                pltpu.make_async_copy(ex_v.at[j], s_sh.at[src_v.at[j]],
                                      dsem).wait()
        return ()
    lax.fori_loop(0, WROWS // 5, sgroup, ())

    plsc.subcore_barrier()
    exd.wait()

    @pl.when(s == 0)
    def _():
        pltpu.sync_copy(s_sh, spart_hbm.at[pl.ds(c * NPAD, NPAD)])


def _sc2_body(src_hbm, ex_hbm, spart_hbm, gamma_hbm,
              src_v, ex_v, s0_v, s1_v, g_v):
    c = lax.axis_index("c")
    s = lax.axis_index("s")
    wid = s * NC + c
    base, _count = _worker_rows(wid)
    base = pl.multiple_of(base, 8)

    pltpu.sync_copy(src_hbm.at[pl.ds(base, WROWS)], src_v)
    pltpu.sync_copy(ex_hbm.at[pl.ds(base, WROWS)], ex_v)
    pltpu.sync_copy(spart_hbm.at[pl.ds(0, NPAD)], s0_v)
    pltpu.sync_copy(spart_hbm.at[pl.ds(NPAD, NPAD)], s1_v)

    @plsc.parallel_loop(0, NPAD // 16, unroll=4)
    def _(i):
        off = pl.ds(pl.multiple_of(i * 16, 16), 16)
        s0_v[off] = s0_v[off] + s1_v[off]

    @plsc.parallel_loop(0, WROWS)
    def _(r):
        for k in range(8):
            sv = src_v[r, pl.ds(k * 16, 16)]
            st = plsc.load_gather(s0_v, [sv])
            g_v[r, pl.ds(k * 16, 16)] = ex_v[r, pl.ds(k * 16, 16)] / st

    pltpu.sync_copy(g_v, gamma_hbm.at[pl.ds(base, WROWS)])


@jax.jit
def kernel(h, e, edge_index, Wn, We, Wa):
    f32 = jnp.float32
    wa_n = Wa[0, DE:].reshape(D, 1)
    w_big = jnp.kron(jnp.eye(8, dtype=f32), We.T)
    wa_t = jnp.tile(Wa[0, :DE], 8).reshape(1, 128)

    e128 = e.reshape(E * DE // 128, 128)
    src1 = edge_index[0].reshape(EROWS, 128)
    gsel = jnp.repeat(jnp.eye(8, dtype=f32), DE, axis=0)
    nblk = 512
    node_feat, n_score, ew128, es8a, src128 = pl.pallas_call(
        _edge_body,
        grid=(NSTEPS,),
        in_specs=[
            pl.BlockSpec((nblk, D), lambda i: (i, 0)),
            pl.BlockSpec((D, D), lambda i: (0, 0)),
            pl.BlockSpec((D, 1), lambda i: (0, 0)),
            pl.BlockSpec((EBLK, 128), lambda i: (i, 0)),
            pl.BlockSpec((128, 128), lambda i: (0, 0)),
            pl.BlockSpec((1, 128), lambda i: (0, 0)),
            pl.BlockSpec((128, 8), lambda i: (0, 0)),
            pl.BlockSpec((ERB, 128), lambda i: (i, 0)),
        ],
        out_specs=[
            pl.BlockSpec((nblk, D), lambda i: (i, 0)),
            pl.BlockSpec((nblk, 1), lambda i: (i, 0)),
            pl.BlockSpec((EBLK, 128), lambda i: (i, 0)),
            pl.BlockSpec((EBLK, 8), lambda i: (i, 0)),
            pl.BlockSpec((ERB, 128), lambda i: (i, 0)),
        ],
        out_shape=[
            jax.ShapeDtypeStruct((N, D), f32),
            jax.ShapeDtypeStruct((N, 1), f32),
            jax.ShapeDtypeStruct((e128.shape[0], 128), f32),
            jax.ShapeDtypeStruct((ERPAD * 16, 8), f32),
            jax.ShapeDtypeStruct((ERPAD, 128), jnp.int32),
        ],
    )(h, Wn, wa_n, e128, w_big, wa_t, gsel, src1)
    es128 = es8a.reshape(ERPAD, 128)
    n_score1 = n_score.reshape(N)

    mesh = plsc.VectorSubcoreMesh(core_axis_name="c", subcore_axis_name="s")
    sc_params = pltpu.CompilerParams(needs_layout_passes=False)

    sc1 = pl.kernel(
        _sc1_body,
        compiler_params=sc_params,
        out_type=(
            jax.ShapeDtypeStruct((ERPAD, 128), f32),
            jax.ShapeDtypeStruct((NC * NPAD,), f32),
        ),
        mesh=mesh,
        scratch_types=[
            pltpu.VMEM((WROWS, 128), jnp.int32),
            pltpu.VMEM((WROWS, 128), f32),
            pltpu.VMEM((N,), f32),
            pltpu.VMEM((WROWS, 128), f32),
            pltpu.VMEM((NPAD,), f32),
            pltpu.VMEM_SHARED((NPAD,), f32),
            pltpu.SemaphoreType.DMA,
            pltpu.SemaphoreType.DMA,
        ],
    )
    ex128, s_part = sc1(src128, es128, n_score1)

    sc2 = pl.kernel(
        _sc2_body,
        compiler_params=sc_params,
        out_type=jax.ShapeDtypeStruct((ERPAD, 128), f32),
        mesh=mesh,
        scratch_types=[
            pltpu.VMEM((WROWS, 128), jnp.int32),
            pltpu.VMEM((WROWS, 128), f32),
            pltpu.VMEM((NPAD,), f32),
            pltpu.VMEM((NPAD,), f32),
            pltpu.VMEM((WROWS, 128), f32),
        ],
    )
    gamma128 = sc2(src128, ex128, s_part)

    gtile = jnp.repeat(jnp.eye(8, dtype=f32), DE, axis=1)
    ewt128 = pl.pallas_call(
        _scale_body,
        grid=(NSTEPS,),
        in_specs=[
            pl.BlockSpec((EBLK, 128), lambda i: (i, 0)),
            pl.BlockSpec((EBLK, 8), lambda i: (i, 0)),
            pl.BlockSpec((8, 128), lambda i: (0, 0)),
        ],
        out_specs=pl.BlockSpec((EBLK, 128), lambda i: (i, 0)),
        out_shape=jax.ShapeDtypeStruct((e128.shape[0], 128), f32),
    )(ew128, gamma128.reshape(ERPAD * 16, 8), gtile)

    e_weighted = ewt128.reshape(E, DE)
    return (node_feat, e_weighted)
